# baseline (speedup 1.0000x reference)
import jax
import jax.numpy as jnp
from jax.experimental import pallas as pl

_T, _D, _F, _E = 2048, 768, 2048, 8
_BM = 1024
_BF = 512


def _dense_body(logits_ref, x_ref, w1_ref, w3_ref, w2_ref, out_ref):
    e = pl.program_id(1)
    f = pl.program_id(2)

    logits = logits_ref[...]
    mx = jnp.max(logits, axis=-1, keepdims=True)
    p = jnp.exp(logits - mx)
    p = p / jnp.sum(p, axis=-1, keepdims=True)
    m1 = jnp.max(p, axis=-1, keepdims=True)
    is1 = p >= m1
    m2 = jnp.max(jnp.where(is1, -1.0, p), axis=-1, keepdims=True)
    sel = jnp.logical_or(is1, p >= m2)
    w = jnp.where(sel, p, 0.0) / (m1 + m2)
    ecol = jax.lax.broadcasted_iota(jnp.int32, w.shape, 1) == e
    w_e = jnp.sum(jnp.where(ecol, w, 0.0), axis=-1)

    x = x_ref[...]
    w1 = w1_ref[0]
    w3 = w3_ref[0]
    w2 = w2_ref[0]
    hi = jax.lax.Precision.HIGHEST
    gate = jax.lax.dot_general(x, w1, (((1,), (1,)), ((), ())), precision=hi)
    up = jax.lax.dot_general(x, w3, (((1,), (1,)), ((), ())), precision=hi)
    h = gate * jax.nn.sigmoid(gate) * up
    y = jax.lax.dot_general(h, w2, (((1,), (1,)), ((), ())), precision=hi)

    @pl.when(jnp.logical_and(e == 0, f == 0))
    def _():
        out_ref[...] = jnp.zeros_like(out_ref)

    out_ref[...] += w_e[:, None] * y


def kernel(x, router_logits, W1, W2, W3):
    grid = (_T // _BM, _E, _F // _BF)
    return pl.pallas_call(
        _dense_body,
        grid=grid,
        in_specs=[
            pl.BlockSpec((_BM, _E), lambda m, e, f: (m, 0)),
            pl.BlockSpec((_BM, _D), lambda m, e, f: (m, 0)),
            pl.BlockSpec((1, _BF, _D), lambda m, e, f: (e, f, 0)),
            pl.BlockSpec((1, _BF, _D), lambda m, e, f: (e, f, 0)),
            pl.BlockSpec((1, _D, _BF), lambda m, e, f: (e, 0, f)),
        ],
        out_specs=pl.BlockSpec((_BM, _D), lambda m, e, f: (m, 0)),
        out_shape=jax.ShapeDtypeStruct((_T, _D), jnp.float32),
    )(router_logits, x, W1, W3, W2)
